# baseline (speedup 1.0000x reference)
import functools

import jax
import jax.numpy as jnp
from jax import lax
from jax.experimental import pallas as pl
from jax.experimental.pallas import tpu as pltpu
from jax.experimental.pallas import tpu_sc as plsc

B = 16
N = 512 * 512
LANES = 16
SHIFT = 18
NB = 1 << (32 - SHIFT)
CHUNK = 2048
NVEC = CHUNK // LANES
NHCHUNK = (N // 2) // CHUNK
UNROLL = 8
NBVEC = NB // LANES
PB = 4096


def _body(logits_hbm, labels_hbm, out_hbm, lbuf, ybuf, hc, ha, obuf,
          pbuf, shared, sem0, sem1):
    cid = lax.axis_index("c")
    sid = lax.axis_index("s")
    wid = cid * 16 + sid
    img = wid // 2
    half = wid % 2

    zeros = jnp.zeros((LANES,), jnp.float32)
    ones = jnp.ones((LANES,), jnp.float32)

    def _zero(i, _):
        hc[pl.ds(i * LANES, LANES)] = zeros
        ha[pl.ds(i * LANES, LANES)] = zeros
        return 0

    lax.fori_loop(0, 2 * NBVEC, _zero, 0)

    sems = (sem0, sem1)

    def _start(c, slot):
        base = img * N + half * (N // 2) + c * CHUNK
        pltpu.async_copy(logits_hbm.at[pl.ds(base, CHUNK)],
                         lbuf.at[slot], sems[slot])
        pltpu.async_copy(labels_hbm.at[pl.ds(base, CHUNK)],
                         ybuf.at[slot], sems[slot])

    def _drain(slot):
        pltpu.make_async_copy(logits_hbm.at[pl.ds(0, CHUNK)],
                              lbuf.at[slot], sems[slot]).wait()
        pltpu.make_async_copy(labels_hbm.at[pl.ds(0, CHUNK)],
                              ybuf.at[slot], sems[slot]).wait()

    def _process(slot):
        def _vec(i, _):
            for u in range(UNROLL):
                off = (i * UNROLL + u) * LANES
                l = lbuf[slot, pl.ds(off, LANES)]
                y = ybuf[slot, pl.ds(off, LANES)]
                yf = y.astype(jnp.float32)
                e = 1.0 - l * (2.0 * yf - 1.0)
                act = jnp.where(e > 0.0, e + 1.0, jnp.exp(e))
                bits = lax.bitcast_convert_type(e, jnp.int32)
                xm = (bits >> 31) | jnp.int32(-(2**31))
                key = bits ^ xm
                bin_ = lax.shift_right_logical(key, SHIFT) + (y << 14)
                plsc.addupdate_scatter(hc, [bin_], ones)
                plsc.addupdate_scatter(ha, [bin_], act)
            return 0

        lax.fori_loop(0, NVEC // UNROLL, _vec, 0)

    _start(0, 0)

    def _chunk2(c2, _):
        _drain(0)
        _start(c2 * 2 + 1, 1)
        _process(0)
        _drain(1)

        @pl.when(c2 * 2 + 2 < NHCHUNK)
        def _():
            _start(c2 * 2 + 2, 0)

        _process(1)
        return 0

    lax.fori_loop(0, NHCHUNK // 2, _chunk2, 0)

    pair = sid // 2

    @pl.when(half == 1)
    def _():
        pltpu.sync_copy(hc, shared.at[pl.ds((pair * 2 + 0) * 2 * NB, 2 * NB)])
        pltpu.sync_copy(ha, shared.at[pl.ds((pair * 2 + 1) * 2 * NB, 2 * NB)])

    plsc.subcore_barrier()

    @pl.when(half == 0)
    def _():
        for k, h in enumerate((hc, ha)):
            for j in range(2 * NB // PB):
                pltpu.sync_copy(
                    shared.at[pl.ds((pair * 2 + k) * 2 * NB + j * PB, PB)],
                    pbuf)

                def _merge(i, _):
                    sl = pl.ds(j * PB + i * LANES, LANES)
                    h[sl] = h[sl] + pbuf[pl.ds(i * LANES, LANES)]
                    return 0

                lax.fori_loop(0, PB // LANES, _merge, 0)

        def _gsum(i, g):
            return g + jnp.sum(hc[pl.ds(NB + i * LANES, LANES)])

        g = lax.fori_loop(0, NBVEC, _gsum, jnp.float32(0.0))
        totn = jnp.float32(N)

        def _scan(i, carry):
            accn, accp, accl = carry
            cneg = hc[pl.ds(i * LANES, LANES)]
            cpos = hc[pl.ds(NB + i * LANES, LANES)]
            sneg = ha[pl.ds(i * LANES, LANES)]
            spos = ha[pl.ds(NB + i * LANES, LANES)]
            n = cneg + cpos
            p = cpos
            s = sneg + spos
            cn = plsc.cumsum(n) + accn
            cp = plsc.cumsum(p) + accp
            n_excl = totn - cn
            p_excl = g - cp
            n_incl = n_excl + n
            p_incl = p_excl + p
            jb = 1.0 - (g - p_incl) / jnp.maximum(g + n_incl - p_incl, 1.0)
            ja = 1.0 - (g - p_excl) / jnp.maximum(g + n_excl - p_excl, 1.0)
            accl = accl + s * (jb - ja) / jnp.maximum(n, 1.0)
            return (accn + jnp.sum(n), accp + jnp.sum(p), accl)

        _, _, accl = lax.fori_loop(
            0, NBVEC, _scan,
            (jnp.float32(0.0), jnp.float32(0.0), jnp.zeros((LANES,), jnp.float32)))

        loss = jnp.sum(accl)
        obuf[...] = jnp.broadcast_to(loss, (LANES,))
        pltpu.sync_copy(obuf, out_hbm.at[img])


@jax.jit
def _lovasz_sc(logits_flat, labels_flat):
    mesh = plsc.VectorSubcoreMesh(core_axis_name="c", subcore_axis_name="s")
    return pl.kernel(
        _body,
        out_type=jax.ShapeDtypeStruct((B, LANES), jnp.float32),
        mesh=mesh,
        compiler_params=pltpu.CompilerParams(needs_layout_passes=False),
        scratch_types=[
            pltpu.VMEM((2, CHUNK), jnp.float32),
            pltpu.VMEM((2, CHUNK), jnp.int32),
            pltpu.VMEM((2 * NB,), jnp.float32),
            pltpu.VMEM((2 * NB,), jnp.float32),
            pltpu.VMEM((LANES,), jnp.float32),
            pltpu.VMEM((PB,), jnp.float32),
            pltpu.VMEM_SHARED((8 * 2 * 2 * NB,), jnp.float32),
            pltpu.SemaphoreType.DMA,
            pltpu.SemaphoreType.DMA,
        ],
    )(logits_flat, labels_flat)


def kernel(logits, labels):
    out = _lovasz_sc(logits.reshape(-1), labels.reshape(-1))
    return jnp.mean(out[:, 0])


# speedup vs baseline: 1.0024x; 1.0024x over previous
import functools

import jax
import jax.numpy as jnp
from jax import lax
from jax.experimental import pallas as pl
from jax.experimental.pallas import tpu as pltpu
from jax.experimental.pallas import tpu_sc as plsc

B = 16
N = 512 * 512
LANES = 16
SHIFT = 18
NB = 1 << (32 - SHIFT)
CHUNK = 2048
NVEC = CHUNK // LANES
NHCHUNK = (N // 2) // CHUNK
UNROLL = 4
NBVEC = NB // LANES
PB = 4096


def _body(logits_hbm, labels_hbm, out_hbm, lbuf, ybuf, hc, ha, obuf,
          pbuf, shared, sem0, sem1):
    cid = lax.axis_index("c")
    sid = lax.axis_index("s")
    wid = cid * 16 + sid
    img = wid // 2
    half = wid % 2

    zeros = jnp.zeros((LANES,), jnp.float32)
    ones = jnp.ones((LANES,), jnp.float32)

    def _zero(i, _):
        hc[pl.ds(i * LANES, LANES)] = zeros
        ha[pl.ds(i * LANES, LANES)] = zeros
        return 0

    lax.fori_loop(0, 2 * NBVEC, _zero, 0)

    sems = (sem0, sem1)

    def _start(c, slot):
        base = img * N + half * (N // 2) + c * CHUNK
        pltpu.async_copy(logits_hbm.at[pl.ds(base, CHUNK)],
                         lbuf.at[slot], sems[slot])
        pltpu.async_copy(labels_hbm.at[pl.ds(base, CHUNK)],
                         ybuf.at[slot], sems[slot])

    def _drain(slot):
        pltpu.make_async_copy(logits_hbm.at[pl.ds(0, CHUNK)],
                              lbuf.at[slot], sems[slot]).wait()
        pltpu.make_async_copy(labels_hbm.at[pl.ds(0, CHUNK)],
                              ybuf.at[slot], sems[slot]).wait()

    def _process(slot):
        def _vec(i, _):
            for u in range(UNROLL):
                off = (i * UNROLL + u) * LANES
                l = lbuf[slot, pl.ds(off, LANES)]
                y = ybuf[slot, pl.ds(off, LANES)]
                yf = y.astype(jnp.float32)
                e = 1.0 - l * (2.0 * yf - 1.0)
                act = jnp.where(e > 0.0, e + 1.0, jnp.exp(e))
                bits = lax.bitcast_convert_type(e, jnp.int32)
                xm = (bits >> 31) | jnp.int32(-(2**31))
                key = bits ^ xm
                bin_ = lax.shift_right_logical(key, SHIFT) + (y << 14)
                plsc.addupdate_scatter(hc, [bin_], ones)
                plsc.addupdate_scatter(ha, [bin_], act)
            return 0

        lax.fori_loop(0, NVEC // UNROLL, _vec, 0)

    _start(0, 0)

    def _chunk2(c2, _):
        _drain(0)
        _start(c2 * 2 + 1, 1)
        _process(0)
        _drain(1)

        @pl.when(c2 * 2 + 2 < NHCHUNK)
        def _():
            _start(c2 * 2 + 2, 0)

        _process(1)
        return 0

    lax.fori_loop(0, NHCHUNK // 2, _chunk2, 0)

    pair = sid // 2

    @pl.when(half == 1)
    def _():
        pltpu.sync_copy(hc, shared.at[pl.ds((pair * 2 + 0) * 2 * NB, 2 * NB)])
        pltpu.sync_copy(ha, shared.at[pl.ds((pair * 2 + 1) * 2 * NB, 2 * NB)])

    plsc.subcore_barrier()

    @pl.when(half == 0)
    def _():
        for k, h in enumerate((hc, ha)):
            for j in range(2 * NB // PB):
                pltpu.sync_copy(
                    shared.at[pl.ds((pair * 2 + k) * 2 * NB + j * PB, PB)],
                    pbuf)

                def _merge(i, _):
                    sl = pl.ds(j * PB + i * LANES, LANES)
                    h[sl] = h[sl] + pbuf[pl.ds(i * LANES, LANES)]
                    return 0

                lax.fori_loop(0, PB // LANES, _merge, 0)

        def _gsum(i, g):
            return g + jnp.sum(hc[pl.ds(NB + i * LANES, LANES)])

        g = lax.fori_loop(0, NBVEC, _gsum, jnp.float32(0.0))
        totn = jnp.float32(N)

        def _scan(i, carry):
            accn, accp, accl = carry
            cneg = hc[pl.ds(i * LANES, LANES)]
            cpos = hc[pl.ds(NB + i * LANES, LANES)]
            sneg = ha[pl.ds(i * LANES, LANES)]
            spos = ha[pl.ds(NB + i * LANES, LANES)]
            n = cneg + cpos
            p = cpos
            s = sneg + spos
            cn = plsc.cumsum(n) + accn
            cp = plsc.cumsum(p) + accp
            n_excl = totn - cn
            p_excl = g - cp
            n_incl = n_excl + n
            p_incl = p_excl + p
            jb = 1.0 - (g - p_incl) / jnp.maximum(g + n_incl - p_incl, 1.0)
            ja = 1.0 - (g - p_excl) / jnp.maximum(g + n_excl - p_excl, 1.0)
            accl = accl + s * (jb - ja) / jnp.maximum(n, 1.0)
            return (accn + jnp.sum(n), accp + jnp.sum(p), accl)

        _, _, accl = lax.fori_loop(
            0, NBVEC, _scan,
            (jnp.float32(0.0), jnp.float32(0.0), jnp.zeros((LANES,), jnp.float32)))

        loss = jnp.sum(accl)
        obuf[...] = jnp.broadcast_to(loss, (LANES,))
        pltpu.sync_copy(obuf, out_hbm.at[img])


@jax.jit
def _lovasz_sc(logits_flat, labels_flat):
    mesh = plsc.VectorSubcoreMesh(core_axis_name="c", subcore_axis_name="s")
    return pl.kernel(
        _body,
        out_type=jax.ShapeDtypeStruct((B, LANES), jnp.float32),
        mesh=mesh,
        compiler_params=pltpu.CompilerParams(needs_layout_passes=False),
        scratch_types=[
            pltpu.VMEM((2, CHUNK), jnp.float32),
            pltpu.VMEM((2, CHUNK), jnp.int32),
            pltpu.VMEM((2 * NB,), jnp.float32),
            pltpu.VMEM((2 * NB,), jnp.float32),
            pltpu.VMEM((LANES,), jnp.float32),
            pltpu.VMEM((PB,), jnp.float32),
            pltpu.VMEM_SHARED((8 * 2 * 2 * NB,), jnp.float32),
            pltpu.SemaphoreType.DMA,
            pltpu.SemaphoreType.DMA,
        ],
    )(logits_flat, labels_flat)


def kernel(logits, labels):
    out = _lovasz_sc(logits.reshape(-1), labels.reshape(-1))
    return jnp.mean(out[:, 0])


# speedup vs baseline: 2.0972x; 2.0922x over previous
import functools

import jax
import jax.numpy as jnp
from jax import lax
from jax.experimental import pallas as pl
from jax.experimental.pallas import tpu as pltpu
from jax.experimental.pallas import tpu_sc as plsc

B = 16
N = 512 * 512
LANES = 16
SHIFT = 18
NB = 1 << (32 - SHIFT)
CHUNK = 2048
NVEC = CHUNK // LANES
NHCHUNK = (N // 2) // CHUNK
UNROLL = 4
NBVEC = NB // LANES
PB = 4096


def _body(logits_hbm, labels_hbm, out_hbm, lbuf, ybuf, hc, ha, obuf,
          pbuf, shared, sem0, sem1):
    cid = lax.axis_index("c")
    sid = lax.axis_index("s")
    wid = cid * 16 + sid
    img = wid // 2
    half = wid % 2

    zeros = jnp.zeros((LANES,), jnp.float32)
    ones = jnp.ones((LANES,), jnp.float32)

    def _zero(i):
        hc[pl.ds(i * LANES, LANES)] = zeros
        ha[pl.ds(i * LANES, LANES)] = zeros

    plsc.parallel_loop(0, 2 * NBVEC, unroll=4)(_zero)

    sems = (sem0, sem1)

    def _start(c, slot):
        base = img * N + half * (N // 2) + c * CHUNK
        pltpu.async_copy(logits_hbm.at[pl.ds(base, CHUNK)],
                         lbuf.at[slot], sems[slot])
        pltpu.async_copy(labels_hbm.at[pl.ds(base, CHUNK)],
                         ybuf.at[slot], sems[slot])

    def _drain(slot):
        pltpu.make_async_copy(logits_hbm.at[pl.ds(0, CHUNK)],
                              lbuf.at[slot], sems[slot]).wait()
        pltpu.make_async_copy(labels_hbm.at[pl.ds(0, CHUNK)],
                              ybuf.at[slot], sems[slot]).wait()

    def _process(slot):
        def _vec(i):
            off = i * LANES
            l = lbuf[slot, pl.ds(off, LANES)]
            y = ybuf[slot, pl.ds(off, LANES)]
            yf = y.astype(jnp.float32)
            e = 1.0 - l * (2.0 * yf - 1.0)
            act = jnp.where(e > 0.0, e + 1.0, jnp.exp(e))
            bits = lax.bitcast_convert_type(e, jnp.int32)
            xm = (bits >> 31) | jnp.int32(-(2**31))
            key = bits ^ xm
            bin_ = lax.shift_right_logical(key, SHIFT) + (y << 14)
            plsc.addupdate_scatter(hc, [bin_], ones)
            plsc.addupdate_scatter(ha, [bin_], act)

        plsc.parallel_loop(0, NVEC, unroll=UNROLL)(_vec)

    _start(0, 0)

    def _chunk2(c2, _):
        _drain(0)
        _start(c2 * 2 + 1, 1)
        _process(0)
        _drain(1)

        @pl.when(c2 * 2 + 2 < NHCHUNK)
        def _():
            _start(c2 * 2 + 2, 0)

        _process(1)
        return 0

    lax.fori_loop(0, NHCHUNK // 2, _chunk2, 0)

    pair = sid // 2

    @pl.when(half == 1)
    def _():
        pltpu.sync_copy(hc, shared.at[pl.ds((pair * 2 + 0) * 2 * NB, 2 * NB)])
        pltpu.sync_copy(ha, shared.at[pl.ds((pair * 2 + 1) * 2 * NB, 2 * NB)])

    plsc.subcore_barrier()

    @pl.when(half == 0)
    def _():
        for k, h in enumerate((hc, ha)):
            for j in range(2 * NB // PB):
                pltpu.sync_copy(
                    shared.at[pl.ds((pair * 2 + k) * 2 * NB + j * PB, PB)],
                    pbuf)

                def _merge(i):
                    sl = pl.ds(j * PB + i * LANES, LANES)
                    h[sl] = h[sl] + pbuf[pl.ds(i * LANES, LANES)]

                plsc.parallel_loop(0, PB // LANES, unroll=4)(_merge)

        def _gsum(i, g):
            return g + jnp.sum(hc[pl.ds(NB + i * LANES, LANES)])

        g = plsc.parallel_loop(0, NBVEC, unroll=4,
                               carry=jnp.float32(0.0))(_gsum)
        totn = jnp.float32(N)

        def _scan(i, carry):
            accn, accp, accl = carry
            cneg = hc[pl.ds(i * LANES, LANES)]
            cpos = hc[pl.ds(NB + i * LANES, LANES)]
            sneg = ha[pl.ds(i * LANES, LANES)]
            spos = ha[pl.ds(NB + i * LANES, LANES)]
            n = cneg + cpos
            p = cpos
            s = sneg + spos
            cn = plsc.cumsum(n) + accn
            cp = plsc.cumsum(p) + accp
            n_excl = totn - cn
            p_excl = g - cp
            n_incl = n_excl + n
            p_incl = p_excl + p
            jb = 1.0 - (g - p_incl) / jnp.maximum(g + n_incl - p_incl, 1.0)
            ja = 1.0 - (g - p_excl) / jnp.maximum(g + n_excl - p_excl, 1.0)
            accl = accl + s * (jb - ja) / jnp.maximum(n, 1.0)
            return (accn + jnp.sum(n), accp + jnp.sum(p), accl)

        _, _, accl = plsc.parallel_loop(
            0, NBVEC, unroll=2,
            carry=(jnp.float32(0.0), jnp.float32(0.0),
                   jnp.zeros((LANES,), jnp.float32)))(
                       lambda i, c: _scan(i, c))

        loss = jnp.sum(accl)
        obuf[...] = jnp.broadcast_to(loss, (LANES,))
        pltpu.sync_copy(obuf, out_hbm.at[img])


@jax.jit
def _lovasz_sc(logits_flat, labels_flat):
    mesh = plsc.VectorSubcoreMesh(core_axis_name="c", subcore_axis_name="s")
    return pl.kernel(
        _body,
        out_type=jax.ShapeDtypeStruct((B, LANES), jnp.float32),
        mesh=mesh,
        compiler_params=pltpu.CompilerParams(needs_layout_passes=False),
        scratch_types=[
            pltpu.VMEM((2, CHUNK), jnp.float32),
            pltpu.VMEM((2, CHUNK), jnp.int32),
            pltpu.VMEM((2 * NB,), jnp.float32),
            pltpu.VMEM((2 * NB,), jnp.float32),
            pltpu.VMEM((LANES,), jnp.float32),
            pltpu.VMEM((PB,), jnp.float32),
            pltpu.VMEM_SHARED((8 * 2 * 2 * NB,), jnp.float32),
            pltpu.SemaphoreType.DMA,
            pltpu.SemaphoreType.DMA,
        ],
    )(logits_flat, labels_flat)


def kernel(logits, labels):
    out = _lovasz_sc(logits.reshape(-1), labels.reshape(-1))
    return jnp.mean(out[:, 0])


# speedup vs baseline: 2.1950x; 1.0467x over previous
import functools

import jax
import jax.numpy as jnp
from jax import lax
from jax.experimental import pallas as pl
from jax.experimental.pallas import tpu as pltpu
from jax.experimental.pallas import tpu_sc as plsc

B = 16
N = 512 * 512
LANES = 16
SHIFT = 18
NB = 1 << (32 - SHIFT)
CHUNK = 4096
NVEC = CHUNK // LANES
NHCHUNK = (N // 2) // CHUNK
UNROLL = 4
NBVEC = NB // LANES
PB = 4096


def _body(logits_hbm, labels_hbm, out_hbm, lbuf, ybuf, hc, ha, obuf,
          pbuf, shared, sem0, sem1):
    cid = lax.axis_index("c")
    sid = lax.axis_index("s")
    wid = cid * 16 + sid
    img = wid // 2
    half = wid % 2

    zeros = jnp.zeros((LANES,), jnp.float32)
    ones = jnp.ones((LANES,), jnp.float32)

    sems = (sem0, sem1)

    def _start(c, slot):
        base = img * N + half * (N // 2) + c * CHUNK
        pltpu.async_copy(logits_hbm.at[pl.ds(base, CHUNK)],
                         lbuf.at[slot], sems[slot])
        pltpu.async_copy(labels_hbm.at[pl.ds(base, CHUNK)],
                         ybuf.at[slot], sems[slot])

    def _drain(slot):
        pltpu.make_async_copy(logits_hbm.at[pl.ds(0, CHUNK)],
                              lbuf.at[slot], sems[slot]).wait()
        pltpu.make_async_copy(labels_hbm.at[pl.ds(0, CHUNK)],
                              ybuf.at[slot], sems[slot]).wait()

    def _process(slot):
        def _vec(i):
            off = i * LANES
            l = lbuf[slot, pl.ds(off, LANES)]
            y = ybuf[slot, pl.ds(off, LANES)]
            yf = y.astype(jnp.float32)
            e = 1.0 - l * (2.0 * yf - 1.0)
            act = jnp.where(e > 0.0, e + 1.0, jnp.exp(e))
            bits = lax.bitcast_convert_type(e, jnp.int32)
            xm = (bits >> 31) | jnp.int32(-(2**31))
            key = bits ^ xm
            bin_ = lax.shift_right_logical(key, SHIFT) + (y << 14)
            plsc.addupdate_scatter(hc, [bin_], ones)
            plsc.addupdate_scatter(ha, [bin_], act)

        plsc.parallel_loop(0, NVEC, unroll=UNROLL)(_vec)

    _start(0, 0)

    def _zero(i):
        hc[pl.ds(i * LANES, LANES)] = zeros
        ha[pl.ds(i * LANES, LANES)] = zeros

    plsc.parallel_loop(0, 2 * NBVEC, unroll=4)(_zero)

    def _chunk2(c2, _):
        _drain(0)
        _start(c2 * 2 + 1, 1)
        _process(0)
        _drain(1)

        @pl.when(c2 * 2 + 2 < NHCHUNK)
        def _():
            _start(c2 * 2 + 2, 0)

        _process(1)
        return 0

    lax.fori_loop(0, NHCHUNK // 2, _chunk2, 0)

    pair = sid // 2

    @pl.when(half == 1)
    def _():
        pltpu.sync_copy(hc, shared.at[pl.ds((pair * 2 + 0) * 2 * NB, 2 * NB)])
        pltpu.sync_copy(ha, shared.at[pl.ds((pair * 2 + 1) * 2 * NB, 2 * NB)])

    plsc.subcore_barrier()

    @pl.when(half == 0)
    def _():
        for k, h in enumerate((hc, ha)):
            for j in range(2 * NB // PB):
                pltpu.sync_copy(
                    shared.at[pl.ds((pair * 2 + k) * 2 * NB + j * PB, PB)],
                    pbuf)

                def _merge(i):
                    sl = pl.ds(j * PB + i * LANES, LANES)
                    h[sl] = h[sl] + pbuf[pl.ds(i * LANES, LANES)]

                plsc.parallel_loop(0, PB // LANES, unroll=4)(_merge)

        def _gsum(i, gv):
            return gv + hc[pl.ds(NB + i * LANES, LANES)]

        gvec = plsc.parallel_loop(0, NBVEC, unroll=4,
                                  carry=zeros)(_gsum)
        g = jnp.sum(gvec)
        totn = jnp.float32(N)
        idx15 = jnp.full((LANES,), 15, jnp.int32)

        def _scan(i, carry):
            accn, accp, accl = carry
            cneg = hc[pl.ds(i * LANES, LANES)]
            cpos = hc[pl.ds(NB + i * LANES, LANES)]
            sneg = ha[pl.ds(i * LANES, LANES)]
            spos = ha[pl.ds(NB + i * LANES, LANES)]
            n = cneg + cpos
            p = cpos
            s = sneg + spos
            cn = plsc.cumsum(n) + accn
            cp = plsc.cumsum(p) + accp
            n_excl = totn - cn
            p_excl = g - cp
            n_incl = n_excl + n
            p_incl = p_excl + p
            jb = 1.0 - (g - p_incl) / jnp.maximum(g + n_incl - p_incl, 1.0)
            ja = 1.0 - (g - p_excl) / jnp.maximum(g + n_excl - p_excl, 1.0)
            accl = accl + s * (jb - ja) / jnp.maximum(n, 1.0)
            return (jnp.take(cn, idx15), jnp.take(cp, idx15), accl)

        _, _, accl = plsc.parallel_loop(
            0, NBVEC, unroll=2,
            carry=(zeros, zeros, zeros))(lambda i, c: _scan(i, c))

        loss = jnp.sum(accl)
        obuf[...] = jnp.broadcast_to(loss, (LANES,))
        pltpu.sync_copy(obuf, out_hbm.at[img])


@jax.jit
def _lovasz_sc(logits_flat, labels_flat):
    mesh = plsc.VectorSubcoreMesh(core_axis_name="c", subcore_axis_name="s")
    return pl.kernel(
        _body,
        out_type=jax.ShapeDtypeStruct((B, LANES), jnp.float32),
        mesh=mesh,
        compiler_params=pltpu.CompilerParams(needs_layout_passes=False),
        scratch_types=[
            pltpu.VMEM((2, CHUNK), jnp.float32),
            pltpu.VMEM((2, CHUNK), jnp.int32),
            pltpu.VMEM((2 * NB,), jnp.float32),
            pltpu.VMEM((2 * NB,), jnp.float32),
            pltpu.VMEM((LANES,), jnp.float32),
            pltpu.VMEM((PB,), jnp.float32),
            pltpu.VMEM_SHARED((8 * 2 * 2 * NB,), jnp.float32),
            pltpu.SemaphoreType.DMA,
            pltpu.SemaphoreType.DMA,
        ],
    )(logits_flat, labels_flat)


def kernel(logits, labels):
    out = _lovasz_sc(logits.reshape(-1), labels.reshape(-1))
    return jnp.mean(out[:, 0])


# speedup vs baseline: 2.2292x; 1.0156x over previous
import functools

import jax
import jax.numpy as jnp
from jax import lax
from jax.experimental import pallas as pl
from jax.experimental.pallas import tpu as pltpu
from jax.experimental.pallas import tpu_sc as plsc

B = 16
N = 512 * 512
LANES = 16
SHIFT = 18
NB = 1 << (32 - SHIFT)
CHUNK = 2048
NVEC = CHUNK // LANES
NHCHUNK = (N // 2) // CHUNK
UNROLL = 4
NBVEC = NB // LANES
PB = 2048
NRING = 4


def _body(logits_hbm, labels_hbm, out_hbm, lbuf, ybuf, hc, ha, obuf,
          pbuf, shared, sem0, sem1, sem2, sem3):
    cid = lax.axis_index("c")
    sid = lax.axis_index("s")
    wid = cid * 16 + sid
    img = wid // 2
    half = wid % 2

    zeros = jnp.zeros((LANES,), jnp.float32)
    ones = jnp.ones((LANES,), jnp.float32)

    sems = (sem0, sem1, sem2, sem3)

    def _start(c, slot):
        base = img * N + half * (N // 2) + c * CHUNK
        pltpu.async_copy(logits_hbm.at[pl.ds(base, CHUNK)],
                         lbuf.at[slot], sems[slot])
        pltpu.async_copy(labels_hbm.at[pl.ds(base, CHUNK)],
                         ybuf.at[slot], sems[slot])

    def _drain(slot):
        pltpu.make_async_copy(logits_hbm.at[pl.ds(0, CHUNK)],
                              lbuf.at[slot], sems[slot]).wait()
        pltpu.make_async_copy(labels_hbm.at[pl.ds(0, CHUNK)],
                              ybuf.at[slot], sems[slot]).wait()

    def _process(slot):
        def _vec(i):
            off = i * LANES
            l = lbuf[slot, pl.ds(off, LANES)]
            y = ybuf[slot, pl.ds(off, LANES)]
            yf = y.astype(jnp.float32)
            e = 1.0 - l * (2.0 * yf - 1.0)
            act = jnp.where(e > 0.0, e + 1.0, jnp.exp(e))
            bits = lax.bitcast_convert_type(e, jnp.int32)
            xm = (bits >> 31) | jnp.int32(-(2**31))
            key = bits ^ xm
            bin_ = lax.shift_right_logical(key, SHIFT) + (y << 14)
            plsc.addupdate_scatter(hc, [bin_], ones)
            plsc.addupdate_scatter(ha, [bin_], act)

        plsc.parallel_loop(0, NVEC, unroll=UNROLL)(_vec)

    _start(0, 0)
    _start(1, 1)

    def _zero(i):
        hc[pl.ds(i * LANES, LANES)] = zeros
        ha[pl.ds(i * LANES, LANES)] = zeros

    plsc.parallel_loop(0, 2 * NBVEC, unroll=4)(_zero)
    _start(2, 2)

    def _chunk4(c4, _):
        for b in range(NRING):
            _drain(b)
            nxt = c4 * NRING + b + (NRING - 1)

            @pl.when(nxt < NHCHUNK)
            def _():
                _start(nxt, (b + NRING - 1) % NRING)

            _process(b)
        return 0

    lax.fori_loop(0, NHCHUNK // NRING, _chunk4, 0)

    pair = sid // 2

    @pl.when(half == 1)
    def _():
        pltpu.sync_copy(hc, shared.at[pl.ds((pair * 2 + 0) * 2 * NB, 2 * NB)])
        pltpu.sync_copy(ha, shared.at[pl.ds((pair * 2 + 1) * 2 * NB, 2 * NB)])

    plsc.subcore_barrier()

    @pl.when(half == 0)
    def _():
        for k, h in enumerate((hc, ha)):
            for j in range(2 * NB // PB):
                pltpu.sync_copy(
                    shared.at[pl.ds((pair * 2 + k) * 2 * NB + j * PB, PB)],
                    pbuf)

                def _merge(i):
                    sl = pl.ds(j * PB + i * LANES, LANES)
                    h[sl] = h[sl] + pbuf[pl.ds(i * LANES, LANES)]

                plsc.parallel_loop(0, PB // LANES, unroll=4)(_merge)

        def _gsum(i, gv):
            return gv + hc[pl.ds(NB + i * LANES, LANES)]

        gvec = plsc.parallel_loop(0, NBVEC, unroll=4,
                                  carry=zeros)(_gsum)
        g = jnp.sum(gvec)
        totn = jnp.float32(N)
        idx15 = jnp.full((LANES,), 15, jnp.int32)

        def _scan(i, carry):
            accn, accp, accl = carry
            cneg = hc[pl.ds(i * LANES, LANES)]
            cpos = hc[pl.ds(NB + i * LANES, LANES)]
            sneg = ha[pl.ds(i * LANES, LANES)]
            spos = ha[pl.ds(NB + i * LANES, LANES)]
            n = cneg + cpos
            p = cpos
            s = sneg + spos
            cn = plsc.cumsum(n) + accn
            cp = plsc.cumsum(p) + accp
            n_excl = totn - cn
            p_excl = g - cp
            n_incl = n_excl + n
            p_incl = p_excl + p
            jb = 1.0 - (g - p_incl) / jnp.maximum(g + n_incl - p_incl, 1.0)
            ja = 1.0 - (g - p_excl) / jnp.maximum(g + n_excl - p_excl, 1.0)
            accl = accl + s * (jb - ja) / jnp.maximum(n, 1.0)
            return (jnp.take(cn, idx15), jnp.take(cp, idx15), accl)

        _, _, accl = plsc.parallel_loop(
            0, NBVEC, unroll=2,
            carry=(zeros, zeros, zeros))(lambda i, c: _scan(i, c))

        loss = jnp.sum(accl)
        obuf[...] = jnp.broadcast_to(loss, (LANES,))
        pltpu.sync_copy(obuf, out_hbm.at[img])


@jax.jit
def _lovasz_sc(logits_flat, labels_flat):
    mesh = plsc.VectorSubcoreMesh(core_axis_name="c", subcore_axis_name="s")
    return pl.kernel(
        _body,
        out_type=jax.ShapeDtypeStruct((B, LANES), jnp.float32),
        mesh=mesh,
        compiler_params=pltpu.CompilerParams(needs_layout_passes=False),
        scratch_types=[
            pltpu.VMEM((NRING, CHUNK), jnp.float32),
            pltpu.VMEM((NRING, CHUNK), jnp.int32),
            pltpu.VMEM((2 * NB,), jnp.float32),
            pltpu.VMEM((2 * NB,), jnp.float32),
            pltpu.VMEM((LANES,), jnp.float32),
            pltpu.VMEM((PB,), jnp.float32),
            pltpu.VMEM_SHARED((8 * 2 * 2 * NB,), jnp.float32),
            pltpu.SemaphoreType.DMA,
            pltpu.SemaphoreType.DMA,
            pltpu.SemaphoreType.DMA,
            pltpu.SemaphoreType.DMA,
        ],
    )(logits_flat, labels_flat)


def kernel(logits, labels):
    out = _lovasz_sc(logits.reshape(-1), labels.reshape(-1))
    return jnp.mean(out[:, 0])
